# Initial kernel scaffold; baseline (speedup 1.0000x reference)
import jax
import jax.numpy as jnp
from jax.experimental import pallas as pl


def kernel(dynamic_emb, emb_rel, W_ih_rel, W_hh_rel, b_ih_rel, b_hh_rel, W_ih_ent, W_hh_ent, b_ih_ent, b_hh_ent, r_to_e, seg_ids):
    raise NotImplementedError("write your pallas kernel here")


# speedup vs baseline: 1.6883x; 1.6883x over previous
import jax
import jax.numpy as jnp
from jax.experimental import pallas as pl

H = 128


def _gru_body(x_ref, w_ref, b_ref, o_ref):
    x = x_ref[...]
    n = jnp.sqrt(jnp.sum(x * x, axis=1, keepdims=True))
    h = x / jnp.maximum(n, 1e-12)
    g = jnp.dot(h, w_ref[...], preferred_element_type=jnp.float32) + b_ref[...]
    r = jax.nn.sigmoid(g[:, 0:H])
    z = jax.nn.sigmoid(g[:, H:2 * H])
    c = jnp.tanh(g[:, 2 * H:3 * H] + r * g[:, 3 * H:4 * H])
    o_ref[...] = (1.0 - z) * c + z * h


def kernel(dynamic_emb, emb_rel, W_ih_rel, W_hh_rel, b_ih_rel, b_hh_rel,
           W_ih_ent, W_hh_ent, b_ih_ent, b_hh_ent, r_to_e, seg_ids):
    N, Hd = dynamic_emb.shape
    W_r = (W_ih_ent[0:H] + W_hh_ent[0:H]).T
    W_z = (W_ih_ent[H:2 * H] + W_hh_ent[H:2 * H]).T
    W_in = W_ih_ent[2 * H:3 * H].T
    W_hn = W_hh_ent[2 * H:3 * H].T
    W = jnp.concatenate([W_r, W_z, W_in, W_hn], axis=1)
    b = jnp.concatenate([
        b_ih_ent[0:2 * H] + b_hh_ent[0:2 * H],
        b_ih_ent[2 * H:3 * H],
        b_hh_ent[2 * H:3 * H]], axis=0)[None, :]

    B = 1000
    out = pl.pallas_call(
        _gru_body,
        grid=(N // B,),
        in_specs=[
            pl.BlockSpec((B, Hd), lambda i: (i, 0)),
            pl.BlockSpec((Hd, 4 * H), lambda i: (0, 0)),
            pl.BlockSpec((1, 4 * H), lambda i: (0, 0)),
        ],
        out_specs=pl.BlockSpec((B, Hd), lambda i: (i, 0)),
        out_shape=jax.ShapeDtypeStruct((N, Hd), jnp.float32),
    )(dynamic_emb, W, b)
    return out
